# Initial kernel scaffold; baseline (speedup 1.0000x reference)
import jax
import jax.numpy as jnp
from jax.experimental import pallas as pl


def kernel(graph, circ_feature_tensor, dis_feature_tensor, association_matrix, train_model, W_circ, W_dis, cheb_W, cheb_b, conv1_w, conv1_b, conv4_w, conv4_b, conv16_w, conv16_b, conv32_w, conv32_b, mlp_w1, mlp_b1, mlp_w2, mlp_b2, mlp_w3, mlp_b3, mlp_w4):
    raise NotImplementedError("write your pallas kernel here")


# speedup vs baseline: 11.1948x; 11.1948x over previous
import functools

import jax
import jax.numpy as jnp
from jax import lax
from jax.experimental import pallas as pl
from jax.experimental.pallas import tpu as pltpu
from jax.experimental.pallas import tpu_sc as plsc

N_CIRC_ = 9984
N_DIS_ = 16
N_ = N_CIRC_ + N_DIS_
E_ = 320000
D_ = 32
K_ = 6

_NC = 2
_NS = 16
_TILES = _NC * _NS
_CHUNK = 128
_NCHUNK = E_ // _CHUNK
_BASE_CH = _NCHUNK // _TILES
_EXTRA = _NCHUNK % _TILES
_RPT = 632
N_P = _NS * _RPT


def _tile_chunks(tile):
    nch = _BASE_CH + jnp.where(tile < _EXTRA, 1, 0)
    base = tile * _BASE_CH + jnp.minimum(tile, _EXTRA)
    return base, nch


def _sc_prop_body(graph_hbm, u_hbm, zeros_hbm, out_hbm,
                  idx_v, rows_v, sem, accum_sh):
    c = lax.axis_index("c")
    s = lax.axis_index("s")
    tile = c * _NS + s
    cbase, nch = _tile_chunks(tile)
    rbase = s * _RPT
    pltpu.sync_copy(zeros_hbm.at[pl.ds(rbase, _RPT)],
                    accum_sh.at[pl.ds(rbase, _RPT)])
    plsc.subcore_barrier()

    def step(i, carry):
        off = (cbase + i) * _CHUNK
        pltpu.sync_copy(graph_hbm.at[:, pl.ds(off, _CHUNK)], idx_v)
        pltpu.async_copy(u_hbm.at[idx_v.at[0]], rows_v, sem).wait()
        pltpu.sync_copy(rows_v, accum_sh.at[idx_v.at[1]], add=True)
        return carry

    lax.fori_loop(0, nch, step, 0)
    plsc.subcore_barrier()
    pltpu.sync_copy(accum_sh.at[pl.ds(rbase, _RPT)],
                    out_hbm.at[c, pl.ds(rbase, _RPT)])


def _sc_deg_body(graph_hbm, ones_hbm, zeros_hbm, out_hbm,
                 idx_v, ones_v, accum_sh):
    c = lax.axis_index("c")
    s = lax.axis_index("s")
    tile = c * _NS + s
    cbase, nch = _tile_chunks(tile)
    rbase = s * _RPT
    pltpu.sync_copy(zeros_hbm.at[pl.ds(rbase, _RPT)],
                    accum_sh.at[pl.ds(rbase, _RPT)])
    pltpu.sync_copy(ones_hbm, ones_v)
    plsc.subcore_barrier()

    def step(i, carry):
        off = (cbase + i) * _CHUNK
        pltpu.sync_copy(graph_hbm.at[:, pl.ds(off, _CHUNK)], idx_v)
        pltpu.sync_copy(ones_v, accum_sh.at[idx_v.at[0]], add=True)
        return carry

    lax.fori_loop(0, nch, step, 0)
    plsc.subcore_barrier()
    pltpu.sync_copy(accum_sh.at[pl.ds(rbase, _RPT)],
                    out_hbm.at[c, pl.ds(rbase, _RPT)])


_SC_MESH = plsc.VectorSubcoreMesh(core_axis_name="c", subcore_axis_name="s")

_SC_PARAMS = pltpu.CompilerParams(use_tc_tiling_on_sc=False)

_sc_prop = pl.kernel(
    _sc_prop_body,
    out_type=jax.ShapeDtypeStruct((_NC, N_P, D_), jnp.float32),
    mesh=_SC_MESH,
    compiler_params=_SC_PARAMS,
    scratch_types=[
        pltpu.VMEM((2, _CHUNK), jnp.int32),
        pltpu.VMEM((_CHUNK, D_), jnp.float32),
        pltpu.SemaphoreType.DMA,
        pltpu.VMEM_SHARED((N_P, D_), jnp.float32),
    ],
)

_sc_deg = pl.kernel(
    _sc_deg_body,
    out_type=jax.ShapeDtypeStruct((_NC, N_P, D_), jnp.float32),
    mesh=_SC_MESH,
    compiler_params=_SC_PARAMS,
    scratch_types=[
        pltpu.VMEM((2, _CHUNK), jnp.int32),
        pltpu.VMEM((_CHUNK, D_), jnp.float32),
        pltpu.VMEM_SHARED((N_P, D_), jnp.float32),
    ],
)


def _mm_body(x_ref, w_ref, o_ref):
    o_ref[...] = jnp.dot(x_ref[...], w_ref[...],
                         preferred_element_type=jnp.float32)


def _matmul(x, w, block_rows):
    m, k = x.shape
    n = w.shape[1]
    grid = m // block_rows
    return pl.pallas_call(
        _mm_body,
        grid=(grid,),
        in_specs=[pl.BlockSpec((block_rows, k), lambda i: (i, 0)),
                  pl.BlockSpec((k, n), lambda i: (0, 0))],
        out_specs=pl.BlockSpec((block_rows, n), lambda i: (i, 0)),
        out_shape=jax.ShapeDtypeStruct((m, n), jnp.float32),
    )(x, w)


def _dinv_u0_body(degp_ref, h_ref, dinv_ref, u0_ref):
    deg = degp_ref[0] + degp_ref[1]
    dinv = jnp.where(deg > 0, lax.rsqrt(jnp.maximum(deg, 1.0)), 0.0)
    dinv_ref[...] = dinv
    u0_ref[...] = dinv * h_ref[...]


def _combine_body(p_ref, dinv_ref, tx0_ref, tx_ref, u_ref, *, alpha):
    dinv = dinv_ref[...]
    tx = -alpha * dinv * (p_ref[0] + p_ref[1]) - tx0_ref[...]
    tx_ref[...] = tx
    u_ref[...] = dinv * tx


def _combine(partials, dinv_b, tx0, alpha):
    two = jax.ShapeDtypeStruct((N_P, D_), jnp.float32)
    return pl.pallas_call(
        functools.partial(_combine_body, alpha=alpha),
        out_shape=(two, two),
    )(partials, dinv_b, tx0)


def _cnn_body(tx_ref, wstk_ref, chb_ref, wcnn_ref, bcnn_ref, cnn_ref):
    res = jnp.dot(tx_ref[...], wstk_ref[...],
                  preferred_element_type=jnp.float32) + chb_ref[...]
    y = jnp.dot(res, wcnn_ref[...],
                preferred_element_type=jnp.float32) + bcnn_ref[...]
    cnn_ref[...] = jnp.maximum(y, 0.0)


def _mlp_body(fc_ref, fd_ref, w1_ref, b1_ref, w2_ref, b2_ref,
              w3_ref, b3_ref, w4_ref, out_ref):
    fc = fc_ref[...]
    cols = []
    for j in range(N_DIS_):
        x = fc * fd_ref[j:j + 1, :]
        h = jnp.dot(x, w1_ref[...], preferred_element_type=jnp.float32)
        h = h + b1_ref[...]
        h = jnp.where(h >= 0, h, 0.01 * h)
        h = jnp.dot(h, w2_ref[...], preferred_element_type=jnp.float32)
        h = h + b2_ref[...]
        h = jnp.where(h >= 0, h, 0.01 * h)
        h = jnp.dot(h, w3_ref[...], preferred_element_type=jnp.float32)
        h = h + b3_ref[...]
        h = jnp.where(h >= 0, h, 0.01 * h)
        p = jnp.dot(h, w4_ref[...], preferred_element_type=jnp.float32)
        cols.append(1.0 / (1.0 + jnp.exp(-p)))
    out_ref[...] = jnp.concatenate(cols, axis=1)


def _build_wcnn(c1w, c4w, c16w, c32w, c1b, c4b, c16b, c32b):
    cols, bs = [], []
    for w_arr, b_arr, w in ((c1w, c1b, 1), (c4w, c4b, 4),
                            (c16w, c16b, 16), (c32w, c32b, 32)):
        ker = w_arr.reshape(6, w)
        p_out = 32 - w + 1
        k_idx = jnp.arange(32)[:, None]
        p_idx = jnp.arange(p_out)[None, :]
        rel = k_idx - p_idx
        valid = (rel >= 0) & (rel < w)
        relc = jnp.clip(rel, 0, w - 1)
        m = jnp.where(valid[None, :, :], ker[:, relc], 0.0)
        m = m.transpose(1, 0, 2).reshape(32, 6 * p_out)
        cols.append(m)
        bs.append(jnp.repeat(b_arr, p_out))
    return jnp.concatenate(cols, axis=1), jnp.concatenate(bs)


def kernel(graph, circ_feature_tensor, dis_feature_tensor, association_matrix,
           train_model, W_circ, W_dis, cheb_W, cheb_b,
           conv1_w, conv1_b, conv4_w, conv4_b, conv16_w, conv16_b,
           conv32_w, conv32_b, mlp_w1, mlp_b1, mlp_w2, mlp_b2,
           mlp_w3, mlp_b3, mlp_w4):
    graph = graph.astype(jnp.int32)
    zeros = jnp.zeros((N_P, D_), jnp.float32)
    ones = jnp.ones((_CHUNK, D_), jnp.float32)

    h_circ = _matmul(circ_feature_tensor, W_circ, 1248)
    h_dis = _matmul(dis_feature_tensor, W_dis, N_DIS_)
    h = jnp.concatenate(
        [h_circ, h_dis, jnp.zeros((N_P - N_, D_), jnp.float32)], axis=0)

    deg_p = _sc_deg(graph, ones, zeros)

    dinv_b, u = pl.pallas_call(
        _dinv_u0_body,
        out_shape=(jax.ShapeDtypeStruct((N_P, D_), jnp.float32),
                   jax.ShapeDtypeStruct((N_P, D_), jnp.float32)),
    )(deg_p, h)

    txs = [h]
    tx_prev = zeros
    alpha = 1.0
    for _ in range(K_ - 1):
        part = _sc_prop(graph, u, zeros)
        tx, u = _combine(part, dinv_b, tx_prev, alpha)
        tx_prev = txs[-1]
        txs.append(tx)
        alpha = 2.0

    tx_all = jnp.concatenate(txs, axis=1)
    wstk = cheb_W.reshape(K_ * D_, D_)
    wcnn, bcnn = _build_wcnn(conv1_w, conv4_w, conv16_w, conv32_w,
                             conv1_b, conv4_b, conv16_b, conv32_b)

    cnn = pl.pallas_call(
        _cnn_body,
        grid=(16,),
        in_specs=[pl.BlockSpec((_RPT, K_ * D_), lambda i: (i, 0)),
                  pl.BlockSpec((K_ * D_, D_), lambda i: (0, 0)),
                  pl.BlockSpec((1, D_), lambda i: (0, 0)),
                  pl.BlockSpec((D_, 474), lambda i: (0, 0)),
                  pl.BlockSpec((1, 474), lambda i: (0, 0))],
        out_specs=pl.BlockSpec((_RPT, 474), lambda i: (i, 0)),
        out_shape=jax.ShapeDtypeStruct((N_P, 474), jnp.float32),
    )(tx_all, wstk, cheb_b.reshape(1, D_), wcnn, bcnn.reshape(1, 474))

    fc = cnn[:N_CIRC_]
    fd = cnn[N_CIRC_:N_]

    pred2d = pl.pallas_call(
        _mlp_body,
        grid=(8,),
        in_specs=[pl.BlockSpec((1248, 474), lambda i: (i, 0)),
                  pl.BlockSpec((N_DIS_, 474), lambda i: (0, 0)),
                  pl.BlockSpec((474, 237), lambda i: (0, 0)),
                  pl.BlockSpec((1, 237), lambda i: (0, 0)),
                  pl.BlockSpec((237, 118), lambda i: (0, 0)),
                  pl.BlockSpec((1, 118), lambda i: (0, 0)),
                  pl.BlockSpec((118, 79), lambda i: (0, 0)),
                  pl.BlockSpec((1, 79), lambda i: (0, 0)),
                  pl.BlockSpec((79, 1), lambda i: (0, 0))],
        out_specs=pl.BlockSpec((1248, N_DIS_), lambda i: (i, 0)),
        out_shape=jax.ShapeDtypeStruct((N_CIRC_, N_DIS_), jnp.float32),
    )(fc, fd, mlp_w1, mlp_b1.reshape(1, 237), mlp_w2, mlp_b2.reshape(1, 118),
      mlp_w3, mlp_b3.reshape(1, 79), mlp_w4)

    pred = pred2d.reshape(N_CIRC_ * N_DIS_, 1)
    labels = association_matrix.reshape(-1, 1)
    return pred, labels, cnn[:N_]
